# baseline (speedup 1.0000x reference)
import functools

import jax
import jax.numpy as jnp
from jax import lax
from jax.experimental import pallas as pl
from jax.experimental.pallas import tpu as pltpu
from jax.experimental.pallas import tpu_sc as plsc

N = 10000
E = 320000
H = 128
G = 64
C = 10
NP = 10240
NT = NP // 128
NC = 2
NS = 16
NW = NC * NS
EPAD = 327680
EPW = EPAD // NW
CHUNK = 128
NCH = EPW // CHUNK
RPT = NP // NS
R = 1024
NB = NP // R
F32 = jnp.float32


SBS = 5
NCHB = NCH // SBS
EPWB = EPW // SBS


def _make_pass(kind):
  gather = kind in ("gather", "gather_deg")
  deg = kind == "gather_deg"
  scalar = kind == "scalar"
  mesh = plsc.VectorSubcoreMesh(core_axis_name="c", subcore_axis_name="s")
  out_type = []
  if gather:
    out_type.append(jax.ShapeDtypeStruct((NC, NP, 128), F32))
  if deg or scalar:
    out_type.append(jax.ShapeDtypeStruct((NW, NP), F32))
  if gather:
    scratch = [
        pltpu.VMEM((EPWB,), jnp.int32),
        pltpu.VMEM((NCHB, CHUNK), jnp.int32),
        pltpu.VMEM((CHUNK, 128), F32),
        pltpu.VMEM((CHUNK, 128), F32),
        pltpu.VMEM_SHARED((NP, 128), F32),
        pltpu.SemaphoreType.DMA,
        pltpu.SemaphoreType.DMA,
        pltpu.SemaphoreType.DMA,
        pltpu.SemaphoreType.DMA,
    ]
    if deg:
      scratch.append(pltpu.VMEM((NP,), F32))
  else:
    scratch = [
        pltpu.VMEM((EPW,), jnp.int32),
        pltpu.VMEM((NCH, CHUNK), jnp.int32),
        pltpu.VMEM((NP,), F32),
        pltpu.VMEM((NP,), F32),
    ]

  @functools.partial(
      pl.kernel, out_type=tuple(out_type), mesh=mesh, scratch_types=scratch,
      compiler_params=pltpu.CompilerParams(needs_layout_passes=False))
  def k(*refs):
    it = iter(refs)
    src_hbm = next(it)
    dst_hbm = next(it)
    table_hbm = next(it) if gather else None
    zeros_hbm = next(it) if gather else None
    stab_hbm = next(it) if scalar else None
    zeros1_hbm = next(it) if (deg or scalar) else None
    out_hbm = next(it) if gather else None
    sout_hbm = next(it) if (deg or scalar) else None
    src_b = next(it)
    dst_b = next(it)
    if gather:
      rows0 = next(it)
      rows1 = next(it)
      acc = next(it)
      sem0 = next(it)
      sem1 = next(it)
      ssem0 = next(it)
      ssem1 = next(it)
    if scalar:
      stab = next(it)
    if deg or scalar:
      sacc = next(it)

    cid = lax.axis_index("c")
    sid = lax.axis_index("s")
    wid = cid * NS + sid
    ebase = wid * EPW
    if gather:
      pltpu.sync_copy(zeros_hbm.at[pl.ds(sid * RPT, RPT)],
                      acc.at[pl.ds(sid * RPT, RPT)])
    if scalar:
      pltpu.sync_copy(stab_hbm, stab)
    if deg or scalar:
      pltpu.sync_copy(zeros1_hbm, sacc)
    if gather:
      plsc.subcore_barrier()

    if gather:
      def start(ch, buf, sem):
        pltpu.async_copy(table_hbm.at[src_b.at[pl.ds(ch * CHUNK, CHUNK)]],
                         buf, sem)

      def wait(buf, sem):
        pltpu.make_async_copy(table_hbm.at[pl.ds(0, CHUNK)], buf, sem).wait()

      def deg_chunk(ch):
        ones = jnp.ones((16,), F32)
        for j in range(CHUNK // 16):
          d_idx = dst_b[ch, pl.ds(j * 16, 16)]
          plsc.addupdate_scatter(sacc, [d_idx], ones)

      def sb_body(sb, carry):
        pltpu.sync_copy(src_hbm.at[pl.ds(ebase + sb * EPWB, EPWB)], src_b)
        pltpu.sync_copy(dst_hbm.at[pl.ds(wid * NCH + sb * NCHB, NCHB)],
                        dst_b)
        start(0, rows0, sem0)
        start(1, rows1, sem1)

        def gbody(g, c):
          ch0 = 2 * g
          if deg:
            deg_chunk(ch0)
            deg_chunk(ch0 + 1)
          wait(rows0, sem0)
          pltpu.async_copy(rows0, acc.at[dst_b.at[ch0]], ssem0, add=True)
          wait(rows1, sem1)
          pltpu.async_copy(rows1, acc.at[dst_b.at[ch0 + 1]], ssem1, add=True)
          wait(rows0, ssem0)

          @pl.when(g < NCHB // 2 - 1)
          def _():
            start(ch0 + 2, rows0, sem0)

          wait(rows1, ssem1)

          @pl.when(g < NCHB // 2 - 1)
          def _():
            start(ch0 + 3, rows1, sem1)

          return c

        lax.fori_loop(0, NCHB // 2, gbody, 0)
        return carry

      lax.fori_loop(0, SBS, sb_body, 0)
    else:
      pltpu.sync_copy(src_hbm.at[pl.ds(ebase, EPW)], src_b)
      pltpu.sync_copy(dst_hbm.at[pl.ds(wid * NCH, NCH)], dst_b)

      def sbody(ch, carry):
        for j in range(CHUNK // 16):
          s_idx = src_b[pl.ds(ch * CHUNK + j * 16, 16)]
          d_idx = dst_b[ch, pl.ds(j * 16, 16)]
          vals = plsc.load_gather(stab, [s_idx])
          plsc.addupdate_scatter(sacc, [d_idx], vals)
        return carry

      lax.fori_loop(0, NCH, sbody, 0)

    if deg or scalar:
      pltpu.sync_copy(sacc, sout_hbm.at[wid])
    if gather:
      plsc.subcore_barrier()
      pltpu.sync_copy(acc.at[pl.ds(sid * RPT, RPT)],
                      out_hbm.at[cid].at[pl.ds(sid * RPT, RPT)])

  return k


@functools.lru_cache(maxsize=None)
def _get_pass(kind):
  return _make_pass(kind)


def _seg128(src_p, dst_p, table, zeros):
  return _get_pass("gather")(src_p, dst_p, table, zeros)[0]


def _seg128_d(src_p, dst_p, table, zeros, zeros1):
  return _get_pass("gather_deg")(src_p, dst_p, table, zeros, zeros1)


def _seg_s(src_p, dst_p, stab, zeros1):
  return _get_pass("scalar")(src_p, dst_p, stab, zeros1)[0]


def _dotT(a, b):
  return lax.dot_general(a, b, (((0,), (0,)), ((), ())),
                         preferred_element_type=F32)


def _dot(a, b):
  return jnp.dot(a, b, preferred_element_type=F32)


def _conv_block(mp_ref, x, wr, wl, b, degc):
  msg = mp_ref[0, :, :H] + mp_ref[1, :, :H]
  agg = msg / degc
  return jax.nn.relu(_dot(agg, wl) + _dot(x, wr) + b)


def _onehot(bc):
  gids = lax.broadcasted_iota(jnp.int32, (1, G), 1)
  return (bc == gids).astype(F32)


def _tc1_body(mp_ref, mpd_ref, x_ref, wr_ref, wl_ref, b_ref, bc_ref, valid_ref,
              h_ref, pool_ref, deg_ref, counts_ref, k_ref, starts_ref):
  i = pl.program_id(0)
  degb = jnp.sum(mpd_ref[...], axis=0)
  degc = jnp.clip(degb, 1.0, None)
  valid = valid_ref[...]
  h = _conv_block(mp_ref, x_ref[...], wr_ref[...], wl_ref[...], b_ref[...],
                  degc) * valid
  h_ref[...] = h
  deg_ref[...] = degb
  oh = _onehot(bc_ref[...]) * valid

  @pl.when(i == 0)
  def _():
    pool_ref[...] = jnp.zeros_like(pool_ref)
    counts_ref[...] = jnp.zeros_like(counts_ref)

  pool_ref[...] += _dotT(oh, h)
  counts_ref[...] += _dotT(oh, valid)

  @pl.when(i == NB - 1)
  def _():
    counts = counts_ref[...]
    k_ref[...] = jnp.ceil(jnp.float32(0.8) * counts)
    rr = lax.broadcasted_iota(jnp.int32, (G, G), 0)
    cc = lax.broadcasted_iota(jnp.int32, (G, G), 1)
    m = (cc < rr).astype(F32)
    starts_ref[...] = _dot(m, counts).astype(jnp.int32)


def _tc1(mp1, mpd, x_pad, wr, wl, b, bc, valid):
  return pl.pallas_call(
      _tc1_body,
      grid=(NB,),
      in_specs=[
          pl.BlockSpec((NC, R, H), lambda i: (0, i, 0)),
          pl.BlockSpec((NW, R, 1), lambda i: (0, i, 0)),
          pl.BlockSpec((R, H), lambda i: (i, 0)),
          pl.BlockSpec((H, H), lambda i: (0, 0)),
          pl.BlockSpec((H, H), lambda i: (0, 0)),
          pl.BlockSpec((1, H), lambda i: (0, 0)),
          pl.BlockSpec((R, 1), lambda i: (i, 0)),
          pl.BlockSpec((R, 1), lambda i: (i, 0)),
      ],
      out_specs=[
          pl.BlockSpec((R, H), lambda i: (i, 0)),
          pl.BlockSpec((G, H), lambda i: (0, 0)),
          pl.BlockSpec((R, 1), lambda i: (i, 0)),
          pl.BlockSpec((G, 1), lambda i: (0, 0)),
          pl.BlockSpec((G, 1), lambda i: (0, 0)),
          pl.BlockSpec((G, 1), lambda i: (0, 0)),
      ],
      out_shape=[
          jax.ShapeDtypeStruct((NP, H), F32),
          jax.ShapeDtypeStruct((G, H), F32),
          jax.ShapeDtypeStruct((NP, 1), F32),
          jax.ShapeDtypeStruct((G, 1), F32),
          jax.ShapeDtypeStruct((G, 1), F32),
          jax.ShapeDtypeStruct((G, 1), jnp.int32),
      ],
  )(mp1, mpd, x_pad, wr, wl, b, bc, valid)


def _tc2_body(mp_ref, h1_ref, deg_ref, wr_ref, wl_ref, b_ref, wsr_ref,
              wsl_ref, bs_ref, bc_ref, valid_ref,
              h2_ref, pool_ref, yrel_ref, yroot_ref):
  i = pl.program_id(0)
  degc = jnp.clip(deg_ref[...], 1.0, None)
  valid = valid_ref[...]
  h2 = _conv_block(mp_ref, h1_ref[...], wr_ref[...], wl_ref[...], b_ref[...],
                   degc) * valid
  h2_ref[...] = h2
  yrel_ref[...] = _dot(h2, wsl_ref[...]) * valid
  yroot_ref[...] = (_dot(h2, wsr_ref[...]) + bs_ref[...]) * valid
  oh = _onehot(bc_ref[...]) * valid

  @pl.when(i == 0)
  def _():
    pool_ref[...] = jnp.zeros_like(pool_ref)

  pool_ref[...] += _dotT(oh, h2)


def _tc2(mp2, h1, deg, wr, wl, b, wsr, wsl, bs, bc, valid):
  return pl.pallas_call(
      _tc2_body,
      grid=(NB,),
      in_specs=[
          pl.BlockSpec((NC, R, H), lambda i: (0, i, 0)),
          pl.BlockSpec((R, H), lambda i: (i, 0)),
          pl.BlockSpec((R, 1), lambda i: (i, 0)),
          pl.BlockSpec((H, H), lambda i: (0, 0)),
          pl.BlockSpec((H, H), lambda i: (0, 0)),
          pl.BlockSpec((1, H), lambda i: (0, 0)),
          pl.BlockSpec((H, 1), lambda i: (0, 0)),
          pl.BlockSpec((H, 1), lambda i: (0, 0)),
          pl.BlockSpec((1, 1), lambda i: (0, 0)),
          pl.BlockSpec((R, 1), lambda i: (i, 0)),
          pl.BlockSpec((R, 1), lambda i: (i, 0)),
      ],
      out_specs=[
          pl.BlockSpec((R, H), lambda i: (i, 0)),
          pl.BlockSpec((G, H), lambda i: (0, 0)),
          pl.BlockSpec((R, 1), lambda i: (i, 0)),
          pl.BlockSpec((R, 1), lambda i: (i, 0)),
      ],
      out_shape=[
          jax.ShapeDtypeStruct((NP, H), F32),
          jax.ShapeDtypeStruct((G, H), F32),
          jax.ShapeDtypeStruct((NP, 1), F32),
          jax.ShapeDtypeStruct((NP, 1), F32),
      ],
  )(mp2, h1, deg, wr, wl, b, wsr, wsl, bs, bc, valid)


def _tc3a_body(mp_ref, yroot_ref, deg_ref, valid_ref, score_ref):
  ssum = jnp.sum(mp_ref[...], axis=0)
  degc = jnp.clip(deg_ref[...], 1.0, None)
  s = jnp.tanh(ssum / degc + yroot_ref[...])
  score_ref[...] = jnp.where(valid_ref[...] > 0, s, jnp.float32(-2.0))


def _tc3a(mp3, yroot, deg, valid):
  return pl.pallas_call(
      _tc3a_body,
      grid=(NB,),
      in_specs=[
          pl.BlockSpec((NW, R, 1), lambda i: (0, i, 0)),
          pl.BlockSpec((R, 1), lambda i: (i, 0)),
          pl.BlockSpec((R, 1), lambda i: (i, 0)),
          pl.BlockSpec((R, 1), lambda i: (i, 0)),
      ],
      out_specs=pl.BlockSpec((R, 1), lambda i: (i, 0)),
      out_shape=jax.ShapeDtypeStruct((NP, 1), F32),
  )(mp3, yroot, deg, valid)


def _tc3b_body(sc_ref, srow_ref, bc_ref, brow_ref, kvec_ref, h2_ref,
               starts_ref, x3_ref, mask_ref, alive_ref):
  i = pl.program_id(0)
  s_i = sc_ref[...]
  b_i = bc_ref[...]
  idx_i = i * 128 + lax.broadcasted_iota(jnp.int32, (128, 1), 0)
  bmin = jnp.min(b_i)
  bmax = jnp.max(b_i)
  jlo = starts_ref[bmin, 0]
  jhi = jnp.where(bmax >= G - 1, N,
                  starts_ref[jnp.minimum(bmax + 1, G - 1), 0])
  tlo = jlo // 128
  thi = (jhi + 127) // 128

  def jbody(t2, rank):
    s_j = srow_ref[pl.ds(t2, 1), :]
    b_j = brow_ref[pl.ds(t2, 1), :]
    idx_j = t2 * 128 + lax.broadcasted_iota(jnp.int32, (1, 128), 1)
    same = b_j == b_i
    better = (s_j > s_i) | ((s_j == s_i) & (idx_j < idx_i))
    contrib = jnp.sum(jnp.where(same & better, 1.0, 0.0), axis=1,
                      keepdims=True)
    return rank + contrib

  rank = lax.fori_loop(tlo, thi, jbody, jnp.zeros((128, 1), F32))
  oh = _onehot(b_i)
  thresh = _dot(oh, kvec_ref[...])
  keep = (rank < thresh).astype(F32)
  x3_ref[...] = h2_ref[...] * keep * s_i
  mask_ref[...] = keep

  @pl.when(i == 0)
  def _():
    alive_ref[...] = jnp.zeros_like(alive_ref)

  alive_ref[...] += _dotT(oh, keep)


def _tc3b(score_col, score_row, bc, brow, starts, kvec, h2):
  return pl.pallas_call(
      _tc3b_body,
      grid=(NT,),
      in_specs=[
          pl.BlockSpec((128, 1), lambda i: (i, 0)),
          pl.BlockSpec((NT, 128), lambda i: (0, 0)),
          pl.BlockSpec((128, 1), lambda i: (i, 0)),
          pl.BlockSpec((NT, 128), lambda i: (0, 0)),
          pl.BlockSpec((G, 1), lambda i: (0, 0)),
          pl.BlockSpec((128, H), lambda i: (i, 0)),
          pl.BlockSpec(memory_space=pltpu.SMEM),
      ],
      out_specs=[
          pl.BlockSpec((128, H), lambda i: (i, 0)),
          pl.BlockSpec((128, 1), lambda i: (i, 0)),
          pl.BlockSpec((G, 1), lambda i: (0, 0)),
      ],
      out_shape=[
          jax.ShapeDtypeStruct((NP, H), F32),
          jax.ShapeDtypeStruct((NP, 1), F32),
          jax.ShapeDtypeStruct((G, 1), F32),
      ],
  )(score_col, score_row, bc, brow, kvec, h2, starts)


def _tc4_body(mp_ref, mpm_ref, x3_ref, wr_ref, wl_ref, b_ref, mask_ref, bc_ref,
              h3_ref, pool_ref, deg3_ref):
  i = pl.program_id(0)
  mask = mask_ref[...]
  masksum = jnp.sum(mpm_ref[...], axis=0)
  deg3 = jnp.clip(mask * masksum, 1.0, None)
  msg = (mp_ref[0, :, :H] + mp_ref[1, :, :H]) * mask
  agg = msg / deg3
  h3 = mask * jax.nn.relu(_dot(agg, wl_ref[...]) + _dot(x3_ref[...],
                                                        wr_ref[...])
                          + b_ref[...])
  h3_ref[...] = h3
  deg3_ref[...] = deg3
  oh = _onehot(bc_ref[...])

  @pl.when(i == 0)
  def _():
    pool_ref[...] = jnp.zeros_like(pool_ref)

  pool_ref[...] += _dotT(oh, h3)


def _tc4(mp4, mpm, x3, wr, wl, b, mask, bc):
  return pl.pallas_call(
      _tc4_body,
      grid=(NB,),
      in_specs=[
          pl.BlockSpec((NC, R, H), lambda i: (0, i, 0)),
          pl.BlockSpec((NW, R, 1), lambda i: (0, i, 0)),
          pl.BlockSpec((R, H), lambda i: (i, 0)),
          pl.BlockSpec((H, H), lambda i: (0, 0)),
          pl.BlockSpec((H, H), lambda i: (0, 0)),
          pl.BlockSpec((1, H), lambda i: (0, 0)),
          pl.BlockSpec((R, 1), lambda i: (i, 0)),
          pl.BlockSpec((R, 1), lambda i: (i, 0)),
      ],
      out_specs=[
          pl.BlockSpec((R, H), lambda i: (i, 0)),
          pl.BlockSpec((G, H), lambda i: (0, 0)),
          pl.BlockSpec((R, 1), lambda i: (i, 0)),
      ],
      out_shape=[
          jax.ShapeDtypeStruct((NP, H), F32),
          jax.ShapeDtypeStruct((G, H), F32),
          jax.ShapeDtypeStruct((NP, 1), F32),
      ],
  )(mp4, mpm, x3, wr, wl, b, mask, bc)


def _tc5_body(mp_ref, h3_ref, wr_ref, wl_ref, b_ref, mask_ref, deg3_ref,
              bc_ref, p1_ref, p2_ref, p3_ref, counts_ref, alive_ref,
              wlin1_ref, blin1_ref, wlin2_ref, blin2_ref,
              out_ref, pool4_ref):
  i = pl.program_id(0)
  mask = mask_ref[...]
  deg3 = deg3_ref[...]
  msg = (mp_ref[0, :, :H] + mp_ref[1, :, :H]) * mask
  agg = msg / deg3
  h4 = mask * jax.nn.relu(_dot(agg, wl_ref[...]) + _dot(h3_ref[...],
                                                        wr_ref[...])
                          + b_ref[...])
  oh = _onehot(bc_ref[...])

  @pl.when(i == 0)
  def _():
    pool4_ref[...] = jnp.zeros_like(pool4_ref)

  pool4_ref[...] += _dotT(oh, h4)

  @pl.when(i == NB - 1)
  def _():
    cf = jnp.clip(counts_ref[...], 1.0, None)
    ca = jnp.clip(alive_ref[...], 1.0, None)
    zs = (p1_ref[...] / cf, p2_ref[...] / cf, p3_ref[...] / ca,
          pool4_ref[...] / ca)
    zz = (_dot(zs[0], wlin1_ref[0]) + _dot(zs[1], wlin1_ref[1])
          + _dot(zs[2], wlin1_ref[2]) + _dot(zs[3], wlin1_ref[3]))
    zz = jax.nn.relu(zz + blin1_ref[...])
    logits = _dot(zz, wlin2_ref[...]) + blin2_ref[...]
    m = jnp.max(logits, axis=1, keepdims=True)
    ex = jnp.exp(logits - m)
    out_ref[...] = logits - m - jnp.log(jnp.sum(ex, axis=1, keepdims=True))


def _tc5(mp5, h3, wr, wl, b, mask, deg3, bc, p1, p2, p3, counts, alive,
         wlin1, blin1, wlin2, blin2):
  return pl.pallas_call(
      _tc5_body,
      grid=(NB,),
      in_specs=[
          pl.BlockSpec((NC, R, H), lambda i: (0, i, 0)),
          pl.BlockSpec((R, H), lambda i: (i, 0)),
          pl.BlockSpec((H, H), lambda i: (0, 0)),
          pl.BlockSpec((H, H), lambda i: (0, 0)),
          pl.BlockSpec((1, H), lambda i: (0, 0)),
          pl.BlockSpec((R, 1), lambda i: (i, 0)),
          pl.BlockSpec((R, 1), lambda i: (i, 0)),
          pl.BlockSpec((R, 1), lambda i: (i, 0)),
          pl.BlockSpec((G, H), lambda i: (0, 0)),
          pl.BlockSpec((G, H), lambda i: (0, 0)),
          pl.BlockSpec((G, H), lambda i: (0, 0)),
          pl.BlockSpec((G, 1), lambda i: (0, 0)),
          pl.BlockSpec((G, 1), lambda i: (0, 0)),
          pl.BlockSpec((4, H, H), lambda i: (0, 0, 0)),
          pl.BlockSpec((1, H), lambda i: (0, 0)),
          pl.BlockSpec((H, H), lambda i: (0, 0)),
          pl.BlockSpec((1, H), lambda i: (0, 0)),
      ],
      out_specs=pl.BlockSpec((G, H), lambda i: (0, 0)),
      out_shape=jax.ShapeDtypeStruct((G, H), F32),
      scratch_shapes=[pltpu.VMEM((G, H), F32)],
  )(mp5, h3, wr, wl, b, mask, deg3, bc, p1, p2, p3, counts, alive,
    wlin1, blin1, wlin2, blin2)


def kernel(x, edge_index, batch, W1_root, W1_rel, b1, W2_root, W2_rel, b2,
           W3_root, W3_rel, b3, W4_root, W4_rel, b4, Ws_root, Ws_rel, bs,
           W_lin1, b_lin1, W_lin2, b_lin2):
  src = edge_index[0]
  dst = edge_index[1]
  pad_idx = jnp.full((EPAD - E,), NP - 1, jnp.int32)
  src_p = jnp.concatenate([src, pad_idx])
  dst_p = jnp.concatenate([dst, pad_idx]).reshape(EPAD // CHUNK, CHUNK)

  valid = (jnp.arange(NP) < N).astype(F32)[:, None]
  bc = jnp.concatenate([batch, jnp.full((NP - N,), G - 1, jnp.int32)])[:, None]
  brow = bc.reshape(NT, 128)
  x_pad = jnp.pad(x, ((0, NP - N), (0, 0)))

  z128 = jnp.zeros((NP, 128), F32)
  z1 = jnp.zeros((NP,), F32)

  b1r = b1.reshape(1, H)
  b2r = b2.reshape(1, H)
  b3r = b3.reshape(1, H)
  b4r = b4.reshape(1, H)
  bsr = bs.reshape(1, 1)
  wlin1 = W_lin1.reshape(4, H, H)
  blin1 = b_lin1.reshape(1, H)
  wlin2 = jnp.pad(W_lin2, ((0, 0), (0, H - C)))
  blin2 = jnp.concatenate([b_lin2,
                           jnp.full((H - C,), -1e30, F32)]).reshape(1, H)

  mp1, mpd = _seg128_d(src_p, dst_p, x_pad, z128, z1)
  h1, pool1s, deg, counts, kvec, starts = _tc1(
      mp1, mpd.reshape(NW, NP, 1), x_pad, W1_root, W1_rel, b1r, bc, valid)

  mp2 = _seg128(src_p, dst_p, h1, z128)
  h2, pool2s, yrel, yroot = _tc2(
      mp2, h1, deg, W2_root, W2_rel, b2r, Ws_root, Ws_rel, bsr, bc, valid)

  mp3 = _seg_s(src_p, dst_p, yrel.reshape(NP), z1)
  score_col = _tc3a(mp3.reshape(NW, NP, 1), yroot, deg, valid)
  score_row = score_col.reshape(NT, 128)

  x3, mask_col, alive = _tc3b(score_col, score_row, bc, brow,
                              starts.reshape(G, 1), kvec, h2)
  mp4 = _seg128(src_p, dst_p, x3, z128)
  mpm = _seg_s(src_p, dst_p, mask_col.reshape(NP), z1)
  h3, pool3s, deg3 = _tc4(mp4, mpm.reshape(NW, NP, 1), x3, W3_root, W3_rel,
                          b3r, mask_col, bc)

  mp5 = _seg128(src_p, dst_p, h3, z128)
  outp = _tc5(mp5, h3, W4_root, W4_rel, b4r, mask_col, deg3, bc,
              pool1s, pool2s, pool3s, counts, alive,
              wlin1, blin1, wlin2, blin2)
  return outp[:, :C]


# speedup vs baseline: 1.0250x; 1.0250x over previous
import functools

import jax
import jax.numpy as jnp
from jax import lax
from jax.experimental import pallas as pl
from jax.experimental.pallas import tpu as pltpu
from jax.experimental.pallas import tpu_sc as plsc

N = 10000
E = 320000
H = 128
G = 64
C = 10
NP = 10240
NT = NP // 128
NC = 2
NS = 16
NW = NC * NS
EPAD = 327680
EPW = EPAD // NW
CHUNK = 128
NCH = EPW // CHUNK
RPT = NP // NS
R = 1024
NB = NP // R
F32 = jnp.float32


SBS = 5
NCHB = NCH // SBS
EPWB = EPW // SBS


def _make_pass(kind):
  gather = kind in ("gather", "gather_deg")
  deg = kind == "gather_deg"
  scalar = kind == "scalar"
  mesh = plsc.VectorSubcoreMesh(core_axis_name="c", subcore_axis_name="s")
  out_type = []
  if gather:
    out_type.append(jax.ShapeDtypeStruct((NC, NP, 128), F32))
  if deg or scalar:
    out_type.append(jax.ShapeDtypeStruct((NW, NP), F32))
  if gather:
    scratch = [
        pltpu.VMEM((EPWB,), jnp.int32),
        pltpu.VMEM((NCHB, CHUNK), jnp.int32),
        pltpu.VMEM((CHUNK, 128), F32),
        pltpu.VMEM((CHUNK, 128), F32),
        pltpu.VMEM_SHARED((NP, 128), F32),
        pltpu.SemaphoreType.DMA,
        pltpu.SemaphoreType.DMA,
    ]
    if deg:
      scratch.append(pltpu.VMEM((NP,), F32))
  else:
    scratch = [
        pltpu.VMEM((EPW,), jnp.int32),
        pltpu.VMEM((NCH, CHUNK), jnp.int32),
        pltpu.VMEM((NP,), F32),
        pltpu.VMEM((NP,), F32),
    ]

  @functools.partial(
      pl.kernel, out_type=tuple(out_type), mesh=mesh, scratch_types=scratch,
      compiler_params=pltpu.CompilerParams(needs_layout_passes=False))
  def k(*refs):
    it = iter(refs)
    src_hbm = next(it)
    dst_hbm = next(it)
    table_hbm = next(it) if gather else None
    zeros_hbm = next(it) if gather else None
    stab_hbm = next(it) if scalar else None
    zeros1_hbm = next(it) if (deg or scalar) else None
    out_hbm = next(it) if gather else None
    sout_hbm = next(it) if (deg or scalar) else None
    src_b = next(it)
    dst_b = next(it)
    if gather:
      rows0 = next(it)
      rows1 = next(it)
      acc = next(it)
      sem0 = next(it)
      sem1 = next(it)
    if scalar:
      stab = next(it)
    if deg or scalar:
      sacc = next(it)

    cid = lax.axis_index("c")
    sid = lax.axis_index("s")
    wid = cid * NS + sid
    ebase = wid * EPW
    if gather:
      pltpu.sync_copy(zeros_hbm.at[pl.ds(0, CHUNK)], rows0)
      for r in range(RPT // CHUNK):
        pltpu.sync_copy(rows0, acc.at[pl.ds(sid * RPT + r * CHUNK, CHUNK)])
    if scalar:
      pltpu.sync_copy(stab_hbm, stab)
    if deg or scalar:
      pltpu.sync_copy(zeros1_hbm, sacc)
    if gather:
      plsc.subcore_barrier()

    if gather:
      def start(ch, buf, sem):
        pltpu.async_copy(table_hbm.at[src_b.at[pl.ds(ch * CHUNK, CHUNK)]],
                         buf, sem)

      def wait(buf, sem):
        pltpu.make_async_copy(table_hbm.at[pl.ds(0, CHUNK)], buf, sem).wait()

      def deg_chunk(ch):
        ones = jnp.ones((16,), F32)
        for j in range(CHUNK // 16):
          d_idx = dst_b[ch, pl.ds(j * 16, 16)]
          plsc.addupdate_scatter(sacc, [d_idx], ones)

      def sb_body(sb, carry):
        pltpu.sync_copy(src_hbm.at[pl.ds(ebase + sb * EPWB, EPWB)], src_b)
        pltpu.sync_copy(dst_hbm.at[pl.ds(wid * NCH + sb * NCHB, NCHB)],
                        dst_b)
        start(0, rows0, sem0)
        start(1, rows1, sem1)

        def gbody(g, c):
          ch0 = 2 * g
          if deg:
            deg_chunk(ch0)
            deg_chunk(ch0 + 1)
          wait(rows0, sem0)
          pltpu.sync_copy(rows0, acc.at[dst_b.at[ch0]], add=True)

          @pl.when(g < NCHB // 2 - 1)
          def _():
            start(ch0 + 2, rows0, sem0)

          wait(rows1, sem1)
          pltpu.sync_copy(rows1, acc.at[dst_b.at[ch0 + 1]], add=True)

          @pl.when(g < NCHB // 2 - 1)
          def _():
            start(ch0 + 3, rows1, sem1)

          return c

        lax.fori_loop(0, NCHB // 2, gbody, 0)
        return carry

      lax.fori_loop(0, SBS, sb_body, 0)
    else:
      pltpu.sync_copy(src_hbm.at[pl.ds(ebase, EPW)], src_b)
      pltpu.sync_copy(dst_hbm.at[pl.ds(wid * NCH, NCH)], dst_b)

      def sbody(ch, carry):
        for j in range(CHUNK // 16):
          s_idx = src_b[pl.ds(ch * CHUNK + j * 16, 16)]
          d_idx = dst_b[ch, pl.ds(j * 16, 16)]
          vals = plsc.load_gather(stab, [s_idx])
          plsc.addupdate_scatter(sacc, [d_idx], vals)
        return carry

      lax.fori_loop(0, NCH, sbody, 0)

    if deg or scalar:
      pltpu.sync_copy(sacc, sout_hbm.at[wid])
    if gather:
      plsc.subcore_barrier()
      pltpu.sync_copy(acc.at[pl.ds(sid * RPT, RPT)],
                      out_hbm.at[cid].at[pl.ds(sid * RPT, RPT)])

  return k


@functools.lru_cache(maxsize=None)
def _get_pass(kind):
  return _make_pass(kind)


def _seg128(src_p, dst_p, table, zeros):
  return _get_pass("gather")(src_p, dst_p, table, zeros)[0]


def _seg128_d(src_p, dst_p, table, zeros, zeros1):
  return _get_pass("gather_deg")(src_p, dst_p, table, zeros, zeros1)


def _seg_s(src_p, dst_p, stab, zeros1):
  return _get_pass("scalar")(src_p, dst_p, stab, zeros1)[0]


def _dotT(a, b):
  return lax.dot_general(a, b, (((0,), (0,)), ((), ())),
                         preferred_element_type=F32)


def _dot(a, b):
  return jnp.dot(a, b, preferred_element_type=F32)


def _conv_block(mp_ref, x, wr, wl, b, degc):
  msg = mp_ref[0, :, :H] + mp_ref[1, :, :H]
  agg = msg / degc
  return jax.nn.relu(_dot(agg, wl) + _dot(x, wr) + b)


def _onehot(bc):
  gids = lax.broadcasted_iota(jnp.int32, (1, G), 1)
  return (bc == gids).astype(F32)


def _tc1_body(mp_ref, mpd_ref, x_ref, wr_ref, wl_ref, b_ref, bc_ref, valid_ref,
              h_ref, pool_ref, deg_ref, counts_ref, k_ref, starts_ref):
  i = pl.program_id(0)
  degb = jnp.sum(mpd_ref[...], axis=0)
  degc = jnp.clip(degb, 1.0, None)
  valid = valid_ref[...]
  h = _conv_block(mp_ref, x_ref[...], wr_ref[...], wl_ref[...], b_ref[...],
                  degc) * valid
  h_ref[...] = h
  deg_ref[...] = degb
  oh = _onehot(bc_ref[...]) * valid

  @pl.when(i == 0)
  def _():
    pool_ref[...] = jnp.zeros_like(pool_ref)
    counts_ref[...] = jnp.zeros_like(counts_ref)

  pool_ref[...] += _dotT(oh, h)
  counts_ref[...] += _dotT(oh, valid)

  @pl.when(i == NB - 1)
  def _():
    counts = counts_ref[...]
    k_ref[...] = jnp.ceil(jnp.float32(0.8) * counts)
    rr = lax.broadcasted_iota(jnp.int32, (G, G), 0)
    cc = lax.broadcasted_iota(jnp.int32, (G, G), 1)
    m = (cc < rr).astype(F32)
    starts_ref[...] = _dot(m, counts).astype(jnp.int32)


def _tc1(mp1, mpd, x_pad, wr, wl, b, bc, valid):
  return pl.pallas_call(
      _tc1_body,
      grid=(NB,),
      in_specs=[
          pl.BlockSpec((NC, R, H), lambda i: (0, i, 0)),
          pl.BlockSpec((NW, R, 1), lambda i: (0, i, 0)),
          pl.BlockSpec((R, H), lambda i: (i, 0)),
          pl.BlockSpec((H, H), lambda i: (0, 0)),
          pl.BlockSpec((H, H), lambda i: (0, 0)),
          pl.BlockSpec((1, H), lambda i: (0, 0)),
          pl.BlockSpec((R, 1), lambda i: (i, 0)),
          pl.BlockSpec((R, 1), lambda i: (i, 0)),
      ],
      out_specs=[
          pl.BlockSpec((R, H), lambda i: (i, 0)),
          pl.BlockSpec((G, H), lambda i: (0, 0)),
          pl.BlockSpec((R, 1), lambda i: (i, 0)),
          pl.BlockSpec((G, 1), lambda i: (0, 0)),
          pl.BlockSpec((G, 1), lambda i: (0, 0)),
          pl.BlockSpec((G, 1), lambda i: (0, 0)),
      ],
      out_shape=[
          jax.ShapeDtypeStruct((NP, H), F32),
          jax.ShapeDtypeStruct((G, H), F32),
          jax.ShapeDtypeStruct((NP, 1), F32),
          jax.ShapeDtypeStruct((G, 1), F32),
          jax.ShapeDtypeStruct((G, 1), F32),
          jax.ShapeDtypeStruct((G, 1), jnp.int32),
      ],
  )(mp1, mpd, x_pad, wr, wl, b, bc, valid)


def _tc2_body(mp_ref, h1_ref, deg_ref, wr_ref, wl_ref, b_ref, wsr_ref,
              wsl_ref, bs_ref, bc_ref, valid_ref,
              h2_ref, pool_ref, yrel_ref, yroot_ref):
  i = pl.program_id(0)
  degc = jnp.clip(deg_ref[...], 1.0, None)
  valid = valid_ref[...]
  h2 = _conv_block(mp_ref, h1_ref[...], wr_ref[...], wl_ref[...], b_ref[...],
                   degc) * valid
  h2_ref[...] = h2
  yrel_ref[...] = _dot(h2, wsl_ref[...]) * valid
  yroot_ref[...] = (_dot(h2, wsr_ref[...]) + bs_ref[...]) * valid
  oh = _onehot(bc_ref[...]) * valid

  @pl.when(i == 0)
  def _():
    pool_ref[...] = jnp.zeros_like(pool_ref)

  pool_ref[...] += _dotT(oh, h2)


def _tc2(mp2, h1, deg, wr, wl, b, wsr, wsl, bs, bc, valid):
  return pl.pallas_call(
      _tc2_body,
      grid=(NB,),
      in_specs=[
          pl.BlockSpec((NC, R, H), lambda i: (0, i, 0)),
          pl.BlockSpec((R, H), lambda i: (i, 0)),
          pl.BlockSpec((R, 1), lambda i: (i, 0)),
          pl.BlockSpec((H, H), lambda i: (0, 0)),
          pl.BlockSpec((H, H), lambda i: (0, 0)),
          pl.BlockSpec((1, H), lambda i: (0, 0)),
          pl.BlockSpec((H, 1), lambda i: (0, 0)),
          pl.BlockSpec((H, 1), lambda i: (0, 0)),
          pl.BlockSpec((1, 1), lambda i: (0, 0)),
          pl.BlockSpec((R, 1), lambda i: (i, 0)),
          pl.BlockSpec((R, 1), lambda i: (i, 0)),
      ],
      out_specs=[
          pl.BlockSpec((R, H), lambda i: (i, 0)),
          pl.BlockSpec((G, H), lambda i: (0, 0)),
          pl.BlockSpec((R, 1), lambda i: (i, 0)),
          pl.BlockSpec((R, 1), lambda i: (i, 0)),
      ],
      out_shape=[
          jax.ShapeDtypeStruct((NP, H), F32),
          jax.ShapeDtypeStruct((G, H), F32),
          jax.ShapeDtypeStruct((NP, 1), F32),
          jax.ShapeDtypeStruct((NP, 1), F32),
      ],
  )(mp2, h1, deg, wr, wl, b, wsr, wsl, bs, bc, valid)


def _tc3a_body(mp_ref, yroot_ref, deg_ref, valid_ref, score_ref):
  ssum = jnp.sum(mp_ref[...], axis=0)
  degc = jnp.clip(deg_ref[...], 1.0, None)
  s = jnp.tanh(ssum / degc + yroot_ref[...])
  score_ref[...] = jnp.where(valid_ref[...] > 0, s, jnp.float32(-2.0))


def _tc3a(mp3, yroot, deg, valid):
  return pl.pallas_call(
      _tc3a_body,
      grid=(NB,),
      in_specs=[
          pl.BlockSpec((NW, R, 1), lambda i: (0, i, 0)),
          pl.BlockSpec((R, 1), lambda i: (i, 0)),
          pl.BlockSpec((R, 1), lambda i: (i, 0)),
          pl.BlockSpec((R, 1), lambda i: (i, 0)),
      ],
      out_specs=pl.BlockSpec((R, 1), lambda i: (i, 0)),
      out_shape=jax.ShapeDtypeStruct((NP, 1), F32),
  )(mp3, yroot, deg, valid)


def _tc3b_body(sc_ref, srow_ref, bc_ref, brow_ref, kvec_ref, h2_ref,
               starts_ref, x3_ref, mask_ref, alive_ref):
  i = pl.program_id(0)
  s_i = sc_ref[...]
  b_i = bc_ref[...]
  idx_i = i * 128 + lax.broadcasted_iota(jnp.int32, (128, 1), 0)
  bmin = jnp.min(b_i)
  bmax = jnp.max(b_i)
  jlo = starts_ref[bmin, 0]
  jhi = jnp.where(bmax >= G - 1, N,
                  starts_ref[jnp.minimum(bmax + 1, G - 1), 0])
  tlo = jlo // 128
  thi = (jhi + 127) // 128

  def jbody(t2, rank):
    s_j = srow_ref[pl.ds(t2, 1), :]
    b_j = brow_ref[pl.ds(t2, 1), :]
    idx_j = t2 * 128 + lax.broadcasted_iota(jnp.int32, (1, 128), 1)
    same = b_j == b_i
    better = (s_j > s_i) | ((s_j == s_i) & (idx_j < idx_i))
    contrib = jnp.sum(jnp.where(same & better, 1.0, 0.0), axis=1,
                      keepdims=True)
    return rank + contrib

  rank = lax.fori_loop(tlo, thi, jbody, jnp.zeros((128, 1), F32))
  oh = _onehot(b_i)
  thresh = _dot(oh, kvec_ref[...])
  keep = (rank < thresh).astype(F32)
  x3_ref[...] = h2_ref[...] * keep * s_i
  mask_ref[...] = keep

  @pl.when(i == 0)
  def _():
    alive_ref[...] = jnp.zeros_like(alive_ref)

  alive_ref[...] += _dotT(oh, keep)


def _tc3b(score_col, score_row, bc, brow, starts, kvec, h2):
  return pl.pallas_call(
      _tc3b_body,
      grid=(NT,),
      in_specs=[
          pl.BlockSpec((128, 1), lambda i: (i, 0)),
          pl.BlockSpec((NT, 128), lambda i: (0, 0)),
          pl.BlockSpec((128, 1), lambda i: (i, 0)),
          pl.BlockSpec((NT, 128), lambda i: (0, 0)),
          pl.BlockSpec((G, 1), lambda i: (0, 0)),
          pl.BlockSpec((128, H), lambda i: (i, 0)),
          pl.BlockSpec(memory_space=pltpu.SMEM),
      ],
      out_specs=[
          pl.BlockSpec((128, H), lambda i: (i, 0)),
          pl.BlockSpec((128, 1), lambda i: (i, 0)),
          pl.BlockSpec((G, 1), lambda i: (0, 0)),
      ],
      out_shape=[
          jax.ShapeDtypeStruct((NP, H), F32),
          jax.ShapeDtypeStruct((NP, 1), F32),
          jax.ShapeDtypeStruct((G, 1), F32),
      ],
  )(score_col, score_row, bc, brow, kvec, h2, starts)


def _tc4_body(mp_ref, mpm_ref, x3_ref, wr_ref, wl_ref, b_ref, mask_ref, bc_ref,
              h3_ref, pool_ref, deg3_ref):
  i = pl.program_id(0)
  mask = mask_ref[...]
  masksum = jnp.sum(mpm_ref[...], axis=0)
  deg3 = jnp.clip(mask * masksum, 1.0, None)
  msg = (mp_ref[0, :, :H] + mp_ref[1, :, :H]) * mask
  agg = msg / deg3
  h3 = mask * jax.nn.relu(_dot(agg, wl_ref[...]) + _dot(x3_ref[...],
                                                        wr_ref[...])
                          + b_ref[...])
  h3_ref[...] = h3
  deg3_ref[...] = deg3
  oh = _onehot(bc_ref[...])

  @pl.when(i == 0)
  def _():
    pool_ref[...] = jnp.zeros_like(pool_ref)

  pool_ref[...] += _dotT(oh, h3)


def _tc4(mp4, mpm, x3, wr, wl, b, mask, bc):
  return pl.pallas_call(
      _tc4_body,
      grid=(NB,),
      in_specs=[
          pl.BlockSpec((NC, R, H), lambda i: (0, i, 0)),
          pl.BlockSpec((NW, R, 1), lambda i: (0, i, 0)),
          pl.BlockSpec((R, H), lambda i: (i, 0)),
          pl.BlockSpec((H, H), lambda i: (0, 0)),
          pl.BlockSpec((H, H), lambda i: (0, 0)),
          pl.BlockSpec((1, H), lambda i: (0, 0)),
          pl.BlockSpec((R, 1), lambda i: (i, 0)),
          pl.BlockSpec((R, 1), lambda i: (i, 0)),
      ],
      out_specs=[
          pl.BlockSpec((R, H), lambda i: (i, 0)),
          pl.BlockSpec((G, H), lambda i: (0, 0)),
          pl.BlockSpec((R, 1), lambda i: (i, 0)),
      ],
      out_shape=[
          jax.ShapeDtypeStruct((NP, H), F32),
          jax.ShapeDtypeStruct((G, H), F32),
          jax.ShapeDtypeStruct((NP, 1), F32),
      ],
  )(mp4, mpm, x3, wr, wl, b, mask, bc)


def _tc5_body(mp_ref, h3_ref, wr_ref, wl_ref, b_ref, mask_ref, deg3_ref,
              bc_ref, p1_ref, p2_ref, p3_ref, counts_ref, alive_ref,
              wlin1_ref, blin1_ref, wlin2_ref, blin2_ref,
              out_ref, pool4_ref):
  i = pl.program_id(0)
  mask = mask_ref[...]
  deg3 = deg3_ref[...]
  msg = (mp_ref[0, :, :H] + mp_ref[1, :, :H]) * mask
  agg = msg / deg3
  h4 = mask * jax.nn.relu(_dot(agg, wl_ref[...]) + _dot(h3_ref[...],
                                                        wr_ref[...])
                          + b_ref[...])
  oh = _onehot(bc_ref[...])

  @pl.when(i == 0)
  def _():
    pool4_ref[...] = jnp.zeros_like(pool4_ref)

  pool4_ref[...] += _dotT(oh, h4)

  @pl.when(i == NB - 1)
  def _():
    cf = jnp.clip(counts_ref[...], 1.0, None)
    ca = jnp.clip(alive_ref[...], 1.0, None)
    zs = (p1_ref[...] / cf, p2_ref[...] / cf, p3_ref[...] / ca,
          pool4_ref[...] / ca)
    zz = (_dot(zs[0], wlin1_ref[0]) + _dot(zs[1], wlin1_ref[1])
          + _dot(zs[2], wlin1_ref[2]) + _dot(zs[3], wlin1_ref[3]))
    zz = jax.nn.relu(zz + blin1_ref[...])
    logits = _dot(zz, wlin2_ref[...]) + blin2_ref[...]
    m = jnp.max(logits, axis=1, keepdims=True)
    ex = jnp.exp(logits - m)
    out_ref[...] = logits - m - jnp.log(jnp.sum(ex, axis=1, keepdims=True))


def _tc5(mp5, h3, wr, wl, b, mask, deg3, bc, p1, p2, p3, counts, alive,
         wlin1, blin1, wlin2, blin2):
  return pl.pallas_call(
      _tc5_body,
      grid=(NB,),
      in_specs=[
          pl.BlockSpec((NC, R, H), lambda i: (0, i, 0)),
          pl.BlockSpec((R, H), lambda i: (i, 0)),
          pl.BlockSpec((H, H), lambda i: (0, 0)),
          pl.BlockSpec((H, H), lambda i: (0, 0)),
          pl.BlockSpec((1, H), lambda i: (0, 0)),
          pl.BlockSpec((R, 1), lambda i: (i, 0)),
          pl.BlockSpec((R, 1), lambda i: (i, 0)),
          pl.BlockSpec((R, 1), lambda i: (i, 0)),
          pl.BlockSpec((G, H), lambda i: (0, 0)),
          pl.BlockSpec((G, H), lambda i: (0, 0)),
          pl.BlockSpec((G, H), lambda i: (0, 0)),
          pl.BlockSpec((G, 1), lambda i: (0, 0)),
          pl.BlockSpec((G, 1), lambda i: (0, 0)),
          pl.BlockSpec((4, H, H), lambda i: (0, 0, 0)),
          pl.BlockSpec((1, H), lambda i: (0, 0)),
          pl.BlockSpec((H, H), lambda i: (0, 0)),
          pl.BlockSpec((1, H), lambda i: (0, 0)),
      ],
      out_specs=pl.BlockSpec((G, H), lambda i: (0, 0)),
      out_shape=jax.ShapeDtypeStruct((G, H), F32),
      scratch_shapes=[pltpu.VMEM((G, H), F32)],
  )(mp5, h3, wr, wl, b, mask, deg3, bc, p1, p2, p3, counts, alive,
    wlin1, blin1, wlin2, blin2)


def kernel(x, edge_index, batch, W1_root, W1_rel, b1, W2_root, W2_rel, b2,
           W3_root, W3_rel, b3, W4_root, W4_rel, b4, Ws_root, Ws_rel, bs,
           W_lin1, b_lin1, W_lin2, b_lin2):
  src = edge_index[0]
  dst = edge_index[1]
  pad_idx = jnp.full((EPAD - E,), NP - 1, jnp.int32)
  src_p = jnp.concatenate([src, pad_idx])
  dst_p = jnp.concatenate([dst, pad_idx]).reshape(EPAD // CHUNK, CHUNK)

  valid = (jnp.arange(NP) < N).astype(F32)[:, None]
  bc = jnp.concatenate([batch, jnp.full((NP - N,), G - 1, jnp.int32)])[:, None]
  brow = bc.reshape(NT, 128)
  x_pad = jnp.pad(x, ((0, NP - N), (0, 0)))

  z128 = jnp.zeros((NP, 128), F32)
  z1 = jnp.zeros((NP,), F32)

  b1r = b1.reshape(1, H)
  b2r = b2.reshape(1, H)
  b3r = b3.reshape(1, H)
  b4r = b4.reshape(1, H)
  bsr = bs.reshape(1, 1)
  wlin1 = W_lin1.reshape(4, H, H)
  blin1 = b_lin1.reshape(1, H)
  wlin2 = jnp.pad(W_lin2, ((0, 0), (0, H - C)))
  blin2 = jnp.concatenate([b_lin2,
                           jnp.full((H - C,), -1e30, F32)]).reshape(1, H)

  mp1, mpd = _seg128_d(src_p, dst_p, x_pad, z128, z1)
  h1, pool1s, deg, counts, kvec, starts = _tc1(
      mp1, mpd.reshape(NW, NP, 1), x_pad, W1_root, W1_rel, b1r, bc, valid)

  mp2 = _seg128(src_p, dst_p, h1, z128)
  h2, pool2s, yrel, yroot = _tc2(
      mp2, h1, deg, W2_root, W2_rel, b2r, Ws_root, Ws_rel, bsr, bc, valid)

  mp3 = _seg_s(src_p, dst_p, yrel.reshape(NP), z1)
  score_col = _tc3a(mp3.reshape(NW, NP, 1), yroot, deg, valid)
  score_row = score_col.reshape(NT, 128)

  x3, mask_col, alive = _tc3b(score_col, score_row, bc, brow,
                              starts.reshape(G, 1), kvec, h2)
  mp4 = _seg128(src_p, dst_p, x3, z128)
  mpm = _seg_s(src_p, dst_p, mask_col.reshape(NP), z1)
  h3, pool3s, deg3 = _tc4(mp4, mpm.reshape(NW, NP, 1), x3, W3_root, W3_rel,
                          b3r, mask_col, bc)

  mp5 = _seg128(src_p, dst_p, h3, z128)
  outp = _tc5(mp5, h3, W4_root, W4_rel, b4r, mask_col, deg3, bc,
              pool1s, pool2s, pool3s, counts, alive,
              wlin1, blin1, wlin2, blin2)
  return outp[:, :C]
